# baseline (speedup 1.0000x reference)
import functools

import jax
import jax.numpy as jnp
from jax import lax
from jax.experimental import pallas as pl
from jax.experimental.pallas import tpu as pltpu
from jax.experimental.pallas import tpu_sc as plsc

_NC = 2
_NS = 16
_NW = _NC * _NS
_CHUNK = 128


def _prep_w0_body(w_ref, o_ref):
    o_ref[...] = jnp.concatenate(
        [w_ref[0], w_ref[1]], axis=1).astype(jnp.bfloat16)


def _prep_w0(w0):
    e, d, h = w0.shape
    return pl.pallas_call(
        _prep_w0_body,
        grid=(e // 2,),
        in_specs=[pl.BlockSpec((2, d, h), lambda i: (i, 0, 0))],
        out_specs=pl.BlockSpec((d, 2 * h), lambda i: (0, i)),
        out_shape=jax.ShapeDtypeStruct((d, e * h), jnp.bfloat16),
    )(w0)


def _pass_a_body(x_ref, w_ref, b_ref, o_ref):
    xb = x_ref[...].astype(jnp.bfloat16)
    acc = jnp.dot(xb, w_ref[...], preferred_element_type=jnp.float32)
    o_ref[...] = jnp.maximum(acc + b_ref[...], 0.0)


def _pass_a(state, w0all, b0all, block_rows=512):
    b, d = state.shape
    eh = w0all.shape[1]
    return pl.pallas_call(
        _pass_a_body,
        grid=(b // block_rows,),
        in_specs=[
            pl.BlockSpec((block_rows, d), lambda i: (i, 0)),
            pl.BlockSpec((d, eh), lambda i: (0, 0)),
            pl.BlockSpec((1, eh), lambda i: (0, 0)),
        ],
        out_specs=pl.BlockSpec((block_rows, eh), lambda i: (i, 0)),
        out_shape=jax.ShapeDtypeStruct((b, eh), jnp.float32),
    )(state, w0all, b0all)


def _sc_dispatch(h0_rows, idx_src, idx_dst, p_rows):
    nw, k, c = idx_src.shape
    per_w = k * c
    width = h0_rows.shape[1]
    mesh = plsc.VectorSubcoreMesh(core_axis_name="c", subcore_axis_name="s")

    @functools.partial(
        pl.kernel,
        mesh=mesh,
        out_type=jax.ShapeDtypeStruct((p_rows, width), jnp.float32),
        scratch_types=[
            pltpu.VMEM((k, c), jnp.int32),
            pltpu.VMEM((k, c), jnp.int32),
            pltpu.VMEM((per_w, width), jnp.float32),
            pltpu.SemaphoreType.DMA,
        ],
    )
    def kern(h0_hbm, isrc_hbm, idst_hbm, xpad_hbm, isrc_v, idst_v, rows_v, sem):
        wid = lax.axis_index("s") * _NC + lax.axis_index("c")
        pltpu.sync_copy(isrc_hbm.at[wid], isrc_v)
        pltpu.sync_copy(idst_hbm.at[wid], idst_v)
        gathers = []
        for j in range(k):
            gathers.append(pltpu.async_copy(
                h0_hbm.at[isrc_v.at[j]],
                rows_v.at[pl.ds(j * c, c)], sem))
        scatters = []
        for j in range(k):
            gathers[j].wait()
            scatters.append(pltpu.async_copy(
                rows_v.at[pl.ds(j * c, c)],
                xpad_hbm.at[idst_v.at[j]], sem))
        for s in scatters:
            s.wait()

    return kern(h0_rows, idx_src, idx_dst)


def _sc_collect(y_pad, idx, b_rows):
    nw, k, c = idx.shape
    per_w = k * c
    width = y_pad.shape[1]
    mesh = plsc.VectorSubcoreMesh(core_axis_name="c", subcore_axis_name="s")

    @functools.partial(
        pl.kernel,
        mesh=mesh,
        out_type=jax.ShapeDtypeStruct((b_rows, width), jnp.float32),
        scratch_types=[
            pltpu.VMEM((k, c), jnp.int32),
            pltpu.VMEM((per_w, width), jnp.float32),
            pltpu.SemaphoreType.DMA,
        ],
    )
    def kern(ypad_hbm, idx_hbm, out_hbm, idx_v, rows_v, sem):
        wid = lax.axis_index("s") * _NC + lax.axis_index("c")
        pltpu.sync_copy(idx_hbm.at[wid], idx_v)
        gathers = []
        for j in range(k):
            gathers.append(pltpu.async_copy(
                ypad_hbm.at[idx_v.at[j]],
                rows_v.at[pl.ds(j * c, c)], sem))
        for g in gathers:
            g.wait()
        pltpu.sync_copy(rows_v, out_hbm.at[pl.ds(wid * per_w, per_w)])

    return kern(y_pad, idx)


def _pass_b_body(se_ref, x_ref, w1_ref, w2_ref, w3_ref, w4_ref, w5_ref,
                 bt_ref, o_ref):
    t = pl.program_id(0)
    par = se_ref[t] % 2
    x = x_ref[...]
    h = jnp.where(par == 0, x[:, :64], x[:, 64:]).astype(jnp.bfloat16)
    for l, w_ref in enumerate((w1_ref, w2_ref, w3_ref, w4_ref)):
        acc = jnp.dot(h, w_ref[0], preferred_element_type=jnp.float32)
        h = jnp.maximum(acc + bt_ref[0, l, :], 0.0).astype(jnp.bfloat16)
    y = (jnp.dot(h, w5_ref[0], preferred_element_type=jnp.float32)
         + bt_ref[0, 4, :])
    o_ref[:, :64] = y
    o_ref[:, 64:] = y


def _pass_b(tile_expert, x_pad, ws_bf, btile, tile_rows, n_tiles, h, a):
    w_spec = pl.BlockSpec((1, h, h), lambda t, se: (se[t], 0, 0))
    grid_spec = pltpu.PrefetchScalarGridSpec(
        num_scalar_prefetch=1,
        grid=(n_tiles,),
        in_specs=[
            pl.BlockSpec((tile_rows, 2 * h), lambda t, se: (t, 0)),
            w_spec, w_spec, w_spec, w_spec,
            pl.BlockSpec((1, h, a), lambda t, se: (se[t], 0, 0)),
            pl.BlockSpec((1, 8, a), lambda t, se: (t, 0, 0)),
        ],
        out_specs=pl.BlockSpec((tile_rows, 2 * a), lambda t, se: (t, 0)),
    )
    return pl.pallas_call(
        _pass_b_body,
        grid_spec=grid_spec,
        out_shape=jax.ShapeDtypeStruct((n_tiles * tile_rows, 2 * a),
                                       jnp.float32),
    )(tile_expert, x_pad, *ws_bf, btile)


def kernel(state, rm_state, W0, b0, W1, b1, W2, b2, W3, b3, W4, b4, W5, b5):
    B, D = state.shape
    E, _, H = W0.shape
    A = W5.shape[2]
    T = 512
    NT = B // T + E
    P = NT * T

    e = rm_state.astype(jnp.int32)
    oh = (e[:, None] == jnp.arange(E, dtype=jnp.int32)[None, :]).astype(jnp.int32)
    cs = jnp.cumsum(oh, axis=0)
    cnt = cs[-1]
    occ = jnp.sum((cs - oh) * oh, axis=1)
    tiles_e = (cnt + T - 1) // T
    tile_start = jnp.concatenate(
        [jnp.zeros((1,), jnp.int32), jnp.cumsum(tiles_e)[:-1].astype(jnp.int32)])
    row_start = tile_start * T
    p = jnp.sum(oh * row_start[None, :], axis=1) + occ
    idx_src = (jnp.arange(B, dtype=jnp.int32) * (E // 2)
               + e // 2).reshape(_NW, -1, _CHUNK)
    idx_dst = p.reshape(_NW, -1, _CHUNK)
    tile_expert = (jnp.sum(
        (jnp.arange(NT, dtype=jnp.int32)[:, None] >= tile_start[None, :])
        .astype(jnp.int32), axis=1) - 1)

    w0all = _prep_w0(W0)
    b0all = b0.reshape(1, E * H)
    ws_bf = tuple(w.astype(jnp.bfloat16) for w in (W1, W2, W3, W4, W5))
    bstack = jnp.stack((b1, b2, b3, b4, b5), axis=1)
    bstack = jnp.pad(bstack, ((0, 0), (0, 3), (0, 0)))
    oh_t = (tile_expert[:, None] == jnp.arange(E, dtype=jnp.int32)[None, :])
    btile = jnp.einsum('te,ela->tla', oh_t.astype(jnp.float32),
                       bstack)

    h0 = _pass_a(state, w0all, b0all)
    h0_rows = h0.reshape(B * E // 2, 2 * H)
    x_pad = _sc_dispatch(h0_rows, idx_src, idx_dst, P)
    y_pad = _pass_b(tile_expert, x_pad, ws_bf, btile, T, NT, H, A)
    wide = _sc_collect(y_pad, idx_dst, B)
    return wide[:, :A]


# speedup vs baseline: 1.1001x; 1.1001x over previous
import functools

import jax
import jax.numpy as jnp
from jax import lax
from jax.experimental import pallas as pl
from jax.experimental.pallas import tpu as pltpu
from jax.experimental.pallas import tpu_sc as plsc

_NC = 2
_NS = 16
_NW = _NC * _NS
_CHUNK = 128


def _prep_w0_body(w_ref, o_ref):
    o_ref[...] = jnp.concatenate(
        [w_ref[0], w_ref[1]], axis=1).astype(jnp.bfloat16)


def _prep_w0(w0):
    e, d, h = w0.shape
    return pl.pallas_call(
        _prep_w0_body,
        grid=(e // 2,),
        in_specs=[pl.BlockSpec((2, d, h), lambda i: (i, 0, 0))],
        out_specs=pl.BlockSpec((d, 2 * h), lambda i: (0, i)),
        out_shape=jax.ShapeDtypeStruct((d, e * h), jnp.bfloat16),
    )(w0)


def _pass_a_body(x_ref, w_ref, b_ref, e_ref, o_ref):
    xb = x_ref[...].astype(jnp.bfloat16)
    acc = jnp.dot(xb, w_ref[...], preferred_element_type=jnp.float32)
    h = jnp.maximum(acc + b_ref[...], 0.0)
    ev = e_ref[...]
    sel = h[:, :64] * (ev == 0.0)
    for k in range(1, 8):
        sel = sel + h[:, 64 * k:64 * (k + 1)] * (ev == float(k))
    o_ref[:, :64] = sel
    o_ref[:, 64:] = sel


def _pass_a(state, w0all, b0all, e2d, block_rows=512):
    b, d = state.shape
    eh = w0all.shape[1]
    return pl.pallas_call(
        _pass_a_body,
        grid=(b // block_rows,),
        in_specs=[
            pl.BlockSpec((block_rows, d), lambda i: (i, 0)),
            pl.BlockSpec((d, eh), lambda i: (0, 0)),
            pl.BlockSpec((1, eh), lambda i: (0, 0)),
            pl.BlockSpec((block_rows, 1), lambda i: (i, 0)),
        ],
        out_specs=pl.BlockSpec((block_rows, 128), lambda i: (i, 0)),
        out_shape=jax.ShapeDtypeStruct((b, 128), jnp.float32),
    )(state, w0all, b0all, e2d)


def _sc_dispatch(h0sel, idx_dst, p_rows):
    nw, k, c = idx_dst.shape
    per_w = k * c
    width = h0sel.shape[1]
    mesh = plsc.VectorSubcoreMesh(core_axis_name="c", subcore_axis_name="s")

    @functools.partial(
        pl.kernel,
        mesh=mesh,
        out_type=jax.ShapeDtypeStruct((p_rows, width), jnp.float32),
        scratch_types=[
            pltpu.VMEM((k, c), jnp.int32),
            pltpu.VMEM((per_w, width), jnp.float32),
            pltpu.SemaphoreType.DMA,
        ],
    )
    def kern(h0_hbm, idst_hbm, xpad_hbm, idst_v, rows_v, sem):
        wid = lax.axis_index("s") * _NC + lax.axis_index("c")
        pltpu.sync_copy(idst_hbm.at[wid], idst_v)
        pltpu.sync_copy(h0_hbm.at[pl.ds(wid * per_w, per_w)], rows_v)
        scatters = []
        for j in range(k):
            scatters.append(pltpu.async_copy(
                rows_v.at[pl.ds(j * c, c)],
                xpad_hbm.at[idst_v.at[j]], sem))
        for s in scatters:
            s.wait()

    return kern(h0sel, idx_dst)


def _sc_collect(y_pad, idx, b_rows):
    nw, k, c = idx.shape
    per_w = k * c
    width = y_pad.shape[1]
    mesh = plsc.VectorSubcoreMesh(core_axis_name="c", subcore_axis_name="s")

    @functools.partial(
        pl.kernel,
        mesh=mesh,
        out_type=jax.ShapeDtypeStruct((b_rows, width), jnp.float32),
        scratch_types=[
            pltpu.VMEM((k, c), jnp.int32),
            pltpu.VMEM((per_w, width), jnp.float32),
            pltpu.SemaphoreType.DMA,
        ],
    )
    def kern(ypad_hbm, idx_hbm, out_hbm, idx_v, rows_v, sem):
        wid = lax.axis_index("s") * _NC + lax.axis_index("c")
        pltpu.sync_copy(idx_hbm.at[wid], idx_v)
        gathers = []
        for j in range(k):
            gathers.append(pltpu.async_copy(
                ypad_hbm.at[idx_v.at[j]],
                rows_v.at[pl.ds(j * c, c)], sem))
        for g in gathers:
            g.wait()
        pltpu.sync_copy(rows_v, out_hbm.at[pl.ds(wid * per_w, per_w)])

    return kern(y_pad, idx)


def _pass_b_body(x_ref, w_ref, b_ref, o_ref):
    x4 = jnp.concatenate(
        [x_ref[q][:, :64] for q in range(4)], axis=1)
    h = x4.astype(jnp.bfloat16)
    for l in range(4):
        acc = jnp.dot(h, w_ref[l, 0], preferred_element_type=jnp.float32)
        h = jnp.maximum(acc + b_ref[0, l, :], 0.0).astype(jnp.bfloat16)
    y4 = (jnp.dot(h, w_ref[4, 0], preferred_element_type=jnp.float32)
          + b_ref[0, 4, :])
    for q in range(4):
        o_ref[q, :, :64] = y4[:, 64 * q:64 * (q + 1)]
        o_ref[q, :, 64:] = y4[:, 64 * q:64 * (q + 1)]


def _pass_b(x_pad4, wbd, bbd, tile_rows, n_tiles):
    nq = n_tiles // 4
    return pl.pallas_call(
        _pass_b_body,
        grid=(nq,),
        in_specs=[
            pl.BlockSpec((4, tile_rows, 128), lambda t: (t, 0, 0)),
            pl.BlockSpec((5, 1, 256, 256), lambda t: (0, t, 0, 0)),
            pl.BlockSpec((1, 5, 256), lambda t: (t, 0, 0)),
        ],
        out_specs=pl.BlockSpec((4, tile_rows, 128), lambda t: (t, 0, 0)),
        out_shape=jax.ShapeDtypeStruct((n_tiles, tile_rows, 128),
                                       jnp.float32),
    )(x_pad4, wbd, bbd)


def kernel(state, rm_state, W0, b0, W1, b1, W2, b2, W3, b3, W4, b4, W5, b5):
    B, D = state.shape
    E, _, H = W0.shape
    A = W5.shape[2]
    T = 512
    NT = B // T + E
    NQ = NT // 4
    P = NT * T

    e = rm_state.astype(jnp.int32)
    oh = (e[:, None] == jnp.arange(E, dtype=jnp.int32)[None, :]).astype(jnp.int32)
    cs = jnp.cumsum(oh, axis=0)
    cnt = cs[-1]
    occ = jnp.sum((cs - oh) * oh, axis=1)
    tiles_e = (cnt + T - 1) // T
    tile_start = jnp.concatenate(
        [jnp.zeros((1,), jnp.int32), jnp.cumsum(tiles_e)[:-1].astype(jnp.int32)])
    row_start = tile_start * T
    p = jnp.sum(oh * row_start[None, :], axis=1) + occ
    idx_dst = p.reshape(_NW, -1, _CHUNK)
    tile_expert = (jnp.sum(
        (jnp.arange(NT, dtype=jnp.int32)[:, None] >= tile_start[None, :])
        .astype(jnp.int32), axis=1) - 1)
    e2d = e.astype(jnp.float32).reshape(B, 1)

    wstk = jnp.stack((W1, W2, W3, W4, W5)).astype(jnp.bfloat16)
    wt = wstk[:, tile_expert]
    wt = wt.reshape(5, NQ, 4, H, H)
    wbd = jnp.zeros((5, NQ, 4, H, 4, H), jnp.bfloat16)
    for q in range(4):
        wbd = wbd.at[:, :, q, :, q, :].set(wt[:, :, q])
    wbd = wbd.reshape(5, NQ, 4 * H, 4 * H)
    bstack = jnp.stack((b1, b2, b3, b4, b5), axis=1)
    bt = bstack[tile_expert]
    bbd = bt.reshape(NQ, 4, 5, A).transpose(0, 2, 1, 3).reshape(NQ, 5, 4 * A)

    w0all = _prep_w0(W0)
    b0all = b0.reshape(1, E * H)
    h0sel = _pass_a(state, w0all, b0all, e2d)
    x_pad = _sc_dispatch(h0sel, idx_dst, P)
    x_pad4 = x_pad.reshape(NT, T, 128)
    y_pad4 = _pass_b(x_pad4, wbd, bbd, T, NT)
    y_pad = y_pad4.reshape(P, 128)
    wide = _sc_collect(y_pad, idx_dst, B)
    return wide[:, :A]


# speedup vs baseline: 1.4030x; 1.2754x over previous
import functools

import jax
import jax.numpy as jnp
from jax import lax
from jax.experimental import pallas as pl
from jax.experimental.pallas import tpu as pltpu
from jax.experimental.pallas import tpu_sc as plsc

_NC = 2
_NS = 16
_NW = _NC * _NS
_CHUNK = 128


def _pass_a_body(w0_ref, x_ref, b_ref, e_ref, o_ref, w_scr):
    i = pl.program_id(0)

    @pl.when(i == 0)
    def _():
        w_scr[...] = jnp.concatenate(
            [w0_ref[k] for k in range(8)], axis=1).astype(jnp.bfloat16)

    @pl.when(i > 0)
    def _():
        xb = x_ref[...].astype(jnp.bfloat16)
        acc = jnp.dot(xb, w_scr[...], preferred_element_type=jnp.float32)
        h = jnp.maximum(acc + b_ref[...], 0.0)
        ev = e_ref[...]
        sel = h[:, :64] * (ev == 0.0)
        for k in range(1, 8):
            sel = sel + h[:, 64 * k:64 * (k + 1)] * (ev == float(k))
        o_ref[:, :64] = sel
        o_ref[:, 64:] = sel


def _pass_a(state, w0, b0all, e2d, block_rows=512):
    b, d = state.shape
    e, _, h = w0.shape
    eh = e * h

    def shifted(i):
        return jnp.maximum(i - 1, 0)

    return pl.pallas_call(
        _pass_a_body,
        grid=(b // block_rows + 1,),
        in_specs=[
            pl.BlockSpec((e, d, h), lambda i: (0, 0, 0)),
            pl.BlockSpec((block_rows, d), lambda i: (shifted(i), 0)),
            pl.BlockSpec((1, eh), lambda i: (0, 0)),
            pl.BlockSpec((block_rows, 1), lambda i: (shifted(i), 0)),
        ],
        out_specs=pl.BlockSpec((block_rows, 128), lambda i: (shifted(i), 0)),
        out_shape=jax.ShapeDtypeStruct((b, 128), jnp.float32),
        scratch_shapes=[pltpu.VMEM((d, eh), jnp.bfloat16)],
    )(w0, state, b0all, e2d)


def _sc_dispatch(h0sel, idx_dst, p_rows):
    nw, k, c = idx_dst.shape
    per_w = k * c
    width = h0sel.shape[1]
    mesh = plsc.VectorSubcoreMesh(core_axis_name="c", subcore_axis_name="s")

    @functools.partial(
        pl.kernel,
        mesh=mesh,
        out_type=jax.ShapeDtypeStruct((p_rows, width), jnp.float32),
        scratch_types=[
            pltpu.VMEM((k, c), jnp.int32),
            pltpu.VMEM((per_w, width), jnp.float32),
            pltpu.SemaphoreType.DMA,
        ],
    )
    def kern(h0_hbm, idst_hbm, xpad_hbm, idst_v, rows_v, sem):
        wid = lax.axis_index("s") * _NC + lax.axis_index("c")
        pltpu.sync_copy(idst_hbm.at[wid], idst_v)
        pltpu.sync_copy(h0_hbm.at[pl.ds(wid * per_w, per_w)], rows_v)
        scatters = []
        for j in range(k):
            scatters.append(pltpu.async_copy(
                rows_v.at[pl.ds(j * c, c)],
                xpad_hbm.at[idst_v.at[j]], sem))
        for s in scatters:
            s.wait()

    return kern(h0sel, idx_dst)


def _sc_collect(y_pad, idx, b_rows, a):
    nw, k, c = idx.shape
    per_w = k * c
    width = y_pad.shape[1]
    mesh = plsc.VectorSubcoreMesh(core_axis_name="c", subcore_axis_name="s")

    @functools.partial(
        pl.kernel,
        mesh=mesh,
        out_type=jax.ShapeDtypeStruct((b_rows, width), jnp.float32),
        scratch_types=[
            pltpu.VMEM((k, c), jnp.int32),
            pltpu.VMEM((per_w, width), jnp.float32),
            pltpu.SemaphoreType.DMA,
        ],
    )
    def kern(ypad_hbm, idx_hbm, out_hbm, idx_v, rows_v, sem):
        wid = lax.axis_index("s") * _NC + lax.axis_index("c")
        pltpu.sync_copy(idx_hbm.at[wid], idx_v)
        gathers = []
        for j in range(k):
            gathers.append(pltpu.async_copy(
                ypad_hbm.at[idx_v.at[j]],
                rows_v.at[pl.ds(j * c, c)], sem))
        for g in gathers:
            g.wait()
        pltpu.sync_copy(rows_v, out_hbm.at[pl.ds(wid * per_w, per_w)])

    return kern(y_pad, idx)


def _pass_b_body(x_ref, wt_ref, bt_ref, o_ref):
    zero = jnp.zeros((64, 64), jnp.bfloat16)

    def bd(l):
        rows = []
        for q in range(4):
            pieces = [zero] * 4
            pieces[q] = wt_ref[l, q]
            rows.append(jnp.concatenate(pieces, axis=1))
        return jnp.concatenate(rows, axis=0)

    def bias(l):
        return jnp.concatenate([bt_ref[q, l, :] for q in range(4)])

    x4 = jnp.concatenate(
        [x_ref[q][:, :64] for q in range(4)], axis=1)
    h = x4.astype(jnp.bfloat16)
    for l in range(4):
        acc = jnp.dot(h, bd(l), preferred_element_type=jnp.float32)
        h = jnp.maximum(acc + bias(l), 0.0).astype(jnp.bfloat16)
    y4 = (jnp.dot(h, bd(4), preferred_element_type=jnp.float32)
          + bias(4))
    for q in range(4):
        o_ref[q, :, :64] = y4[:, 64 * q:64 * (q + 1)]
        o_ref[q, :, 64:] = y4[:, 64 * q:64 * (q + 1)]


def _pass_b(x_pad4, wt, bt, tile_rows, n_tiles, h, a):
    nq = n_tiles // 4
    return pl.pallas_call(
        _pass_b_body,
        grid=(nq,),
        in_specs=[
            pl.BlockSpec((4, tile_rows, 128), lambda t: (t, 0, 0)),
            pl.BlockSpec((5, 4, h, h), lambda t: (0, t, 0, 0)),
            pl.BlockSpec((4, 5, a), lambda t: (t, 0, 0)),
        ],
        out_specs=pl.BlockSpec((4, tile_rows, 128), lambda t: (t, 0, 0)),
        out_shape=jax.ShapeDtypeStruct((n_tiles, tile_rows, 128),
                                       jnp.float32),
    )(x_pad4, wt, bt)


def kernel(state, rm_state, W0, b0, W1, b1, W2, b2, W3, b3, W4, b4, W5, b5):
    B, D = state.shape
    E, _, H = W0.shape
    A = W5.shape[2]
    T = 512
    NT = B // T + E
    P = NT * T

    e = rm_state.astype(jnp.int32)
    oh = (e[:, None] == jnp.arange(E, dtype=jnp.int32)[None, :]).astype(jnp.int32)
    cs = jnp.cumsum(oh, axis=0)
    cnt = cs[-1]
    occ = jnp.sum((cs - oh) * oh, axis=1)
    tiles_e = (cnt + T - 1) // T
    tile_start = jnp.concatenate(
        [jnp.zeros((1,), jnp.int32), jnp.cumsum(tiles_e)[:-1].astype(jnp.int32)])
    row_start = tile_start * T
    p = jnp.sum(oh * row_start[None, :], axis=1) + occ
    idx_dst = p.reshape(_NW, -1, _CHUNK)
    tile_expert = (jnp.sum(
        (jnp.arange(NT, dtype=jnp.int32)[:, None] >= tile_start[None, :])
        .astype(jnp.int32), axis=1) - 1)
    e2d = e.astype(jnp.float32).reshape(B, 1)

    wstk = jnp.stack((W1, W2, W3, W4, W5)).astype(jnp.bfloat16)
    wt = wstk[:, tile_expert]
    bstack = jnp.stack((b1, b2, b3, b4, b5), axis=1)
    bt = bstack[tile_expert]

    b0all = b0.reshape(1, E * H)
    h0sel = _pass_a(state, W0, b0all, e2d)
    x_pad = _sc_dispatch(h0sel, idx_dst, P)
    x_pad4 = x_pad.reshape(NT, T, 128)
    y_pad4 = _pass_b(x_pad4, wt, bt, T, NT, H, A)
    y_pad = y_pad4.reshape(P, 128)
    wide = _sc_collect(y_pad, idx_dst, B, A)
    return wide[:, :A]
